# baseline (speedup 1.0000x reference)
import jax
import jax.numpy as jnp
from jax.experimental import pallas as pl
from jax.experimental.pallas import tpu as pltpu

_D = 512
_BLOCK = 2000


def _bilinear_kernel(a_ref, p_ref, n_ref, wt_ref, b_ref, pos_out, neg_out):
    t = jax.lax.dot_general(a_ref[:], wt_ref[:],
                            (((1,), (1,)), ((), ())),
                            preferred_element_type=jnp.float32)
    bias = b_ref[0]
    logit_p = jnp.sum(p_ref[:] * t, axis=1, keepdims=True) + bias
    logit_n = jnp.sum(n_ref[:] * t, axis=1, keepdims=True) + bias
    pos_out[:] = jax.nn.sigmoid(logit_p)
    neg_out[:] = jax.nn.sigmoid(logit_n)


def kernel(anchor_feat, pos_feat, neg_feat, W, b):
    n = anchor_feat.shape[0]
    w_t = W[0]

    grid = (n // _BLOCK,)
    feat_spec = pl.BlockSpec((_BLOCK, _D), lambda i: (i, 0))
    w_spec = pl.BlockSpec((_D, _D), lambda i: (0, 0))
    b_spec = pl.BlockSpec(memory_space=pltpu.SMEM)
    out_spec = pl.BlockSpec((_BLOCK, 1), lambda i: (i, 0))

    sc_pos, sc_neg = pl.pallas_call(
        _bilinear_kernel,
        grid=grid,
        in_specs=[feat_spec, feat_spec, feat_spec, w_spec, b_spec],
        out_specs=[out_spec, out_spec],
        out_shape=[
            jax.ShapeDtypeStruct((n, 1), jnp.float32),
            jax.ShapeDtypeStruct((n, 1), jnp.float32),
        ],
        compiler_params=pltpu.CompilerParams(
            dimension_semantics=("parallel",),
            vmem_limit_bytes=128 * 1024 * 1024,
        ),
    )(anchor_feat, pos_feat, neg_feat, w_t, b)

    return (sc_pos[:, 0], sc_neg[:, 0])


# speedup vs baseline: 1.0334x; 1.0334x over previous
import jax
import jax.numpy as jnp
from jax.experimental import pallas as pl
from jax.experimental.pallas import tpu as pltpu

_D = 512
_BLOCK = 2000


def _bilinear_kernel(a_ref, p_ref, n_ref, w_ref, b_ref, out_ref):
    t = jax.lax.dot_general(a_ref[:], w_ref[:],
                            (((1,), (1,)), ((), ())),
                            preferred_element_type=jnp.float32)
    bias = b_ref[0]
    logit_p = jnp.sum(p_ref[:] * t, axis=1, keepdims=True) + bias
    logit_n = jnp.sum(n_ref[:] * t, axis=1, keepdims=True) + bias
    out_ref[:] = jax.nn.sigmoid(jnp.concatenate([logit_p, logit_n], axis=1))


def kernel(anchor_feat, pos_feat, neg_feat, W, b):
    n = anchor_feat.shape[0]
    w0 = W[0]

    grid = (n // _BLOCK,)
    feat_spec = pl.BlockSpec((_BLOCK, _D), lambda i: (i, 0))
    w_spec = pl.BlockSpec((_D, _D), lambda i: (0, 0))
    b_spec = pl.BlockSpec(memory_space=pltpu.SMEM)
    out_spec = pl.BlockSpec((_BLOCK, 2), lambda i: (i, 0))

    scores = pl.pallas_call(
        _bilinear_kernel,
        grid=grid,
        in_specs=[feat_spec, feat_spec, feat_spec, w_spec, b_spec],
        out_specs=out_spec,
        out_shape=jax.ShapeDtypeStruct((n, 2), jnp.float32),
        compiler_params=pltpu.CompilerParams(
            dimension_semantics=("parallel",),
            vmem_limit_bytes=128 * 1024 * 1024,
        ),
    )(anchor_feat, pos_feat, neg_feat, w0, b)

    return (scores[:, 0], scores[:, 1])


# speedup vs baseline: 1.2901x; 1.2484x over previous
import jax
import jax.numpy as jnp
from jax.experimental import pallas as pl
from jax.experimental.pallas import tpu as pltpu

_D = 512
_BLOCK = 2000


def _bilinear_kernel(a_ref, p_ref, n_ref, w_ref, b_ref, out_ref):
    t = jax.lax.dot_general(a_ref[:], w_ref[:],
                            (((1,), (1,)), ((), ())),
                            preferred_element_type=jnp.float32)
    bias = b_ref[0]
    logit_p = jnp.sum(p_ref[:] * t, axis=1, keepdims=True) + bias
    logit_n = jnp.sum(n_ref[:] * t, axis=1, keepdims=True) + bias
    sig = jax.nn.sigmoid(jnp.concatenate([logit_p, logit_n], axis=1))
    out_ref[:] = jnp.transpose(sig, (1, 0))[None]


def kernel(anchor_feat, pos_feat, neg_feat, W, b):
    n = anchor_feat.shape[0]
    w0 = W[0]
    g = n // _BLOCK

    grid = (g,)
    feat_spec = pl.BlockSpec((_BLOCK, _D), lambda i: (i, 0))
    w_spec = pl.BlockSpec((_D, _D), lambda i: (0, 0))
    b_spec = pl.BlockSpec(memory_space=pltpu.SMEM)
    out_spec = pl.BlockSpec((1, 2, _BLOCK), lambda i: (i, 0, 0))

    scores = pl.pallas_call(
        _bilinear_kernel,
        grid=grid,
        in_specs=[feat_spec, feat_spec, feat_spec, w_spec, b_spec],
        out_specs=out_spec,
        out_shape=jax.ShapeDtypeStruct((g, 2, _BLOCK), jnp.float32),
        compiler_params=pltpu.CompilerParams(
            dimension_semantics=("parallel",),
            vmem_limit_bytes=128 * 1024 * 1024,
        ),
    )(anchor_feat, pos_feat, neg_feat, w0, b)

    return (scores[:, 0, :].reshape(-1), scores[:, 1, :].reshape(-1))
